# Initial kernel scaffold; baseline (speedup 1.0000x reference)
import functools

import jax
import jax.numpy as jnp
from jax import lax
from jax.experimental import pallas as pl
from jax.experimental.pallas import tpu as pltpu
from jax.experimental.pallas import tpu_sc as plsc

N = 10000
D = 128
E = 320000

NC = 2
NS = 16
CHUNK = 128
CPT = 80
CHUNKS_PER_CORE = NS * CPT
TOT_CHUNKS = NC * CHUNKS_PER_CORE
E_PAD = TOT_CHUNKS * CHUNK
NPAD = 10016
ZROWS = NPAD // NS
OROWS = N // NS

_MESH = plsc.VectorSubcoreMesh(core_axis_name="c", subcore_axis_name="s")


def _sc_agg_body(with_deg, x_hbm, src_hbm, dst_hbm, ones_hbm, zagg_hbm,
                 zdeg_hbm, part_hbm, degp_hbm, agg_sh, deg_sh, src_v, dst_v,
                 ones_v, buf, sem):
    c = lax.axis_index("c")
    s = lax.axis_index("s")
    pltpu.sync_copy(zagg_hbm, agg_sh.at[pl.ds(s * ZROWS, ZROWS)])
    if with_deg:
        pltpu.sync_copy(zdeg_hbm, deg_sh.at[pl.ds(s * ZROWS, ZROWS)])
        pltpu.sync_copy(ones_hbm, ones_v)
    base = c * CHUNKS_PER_CORE + s * CPT
    pltpu.sync_copy(src_hbm.at[pl.ds(base, CPT)], src_v)
    pltpu.sync_copy(dst_hbm.at[pl.ds(base, CPT)], dst_v)
    plsc.subcore_barrier()

    def step(i, carry):
        pltpu.async_copy(x_hbm.at[src_v.at[i]], buf, sem).wait()
        if with_deg:
            pltpu.sync_copy(ones_v, deg_sh.at[dst_v.at[i]], add=True)
        pltpu.sync_copy(buf, agg_sh.at[dst_v.at[i]], add=True)
        return carry

    lax.fori_loop(0, CPT, step, 0)
    plsc.subcore_barrier()
    pltpu.sync_copy(agg_sh.at[pl.ds(s * OROWS, OROWS)],
                    part_hbm.at[c, pl.ds(s * OROWS, OROWS)])
    if with_deg:
        pltpu.sync_copy(deg_sh.at[pl.ds(s * OROWS, OROWS)],
                        degp_hbm.at[c, pl.ds(s * OROWS, OROWS)])


def _make_sc_agg(with_deg):
    out_type = [jax.ShapeDtypeStruct((NC, N, D), jnp.float32)]
    if with_deg:
        out_type.append(jax.ShapeDtypeStruct((NC, N, 16), jnp.float32))
    scratch = [
        pltpu.VMEM_SHARED((NPAD, D), jnp.float32),
        pltpu.VMEM_SHARED((NPAD, 16), jnp.float32) if with_deg else None,
        pltpu.VMEM((CPT, CHUNK), jnp.int32),
        pltpu.VMEM((CPT, CHUNK), jnp.int32),
        pltpu.VMEM((CHUNK, 16), jnp.float32) if with_deg else None,
        pltpu.VMEM((CHUNK, D), jnp.float32),
        pltpu.SemaphoreType.DMA,
    ]
    scratch = [sc for sc in scratch if sc is not None]

    if with_deg:
        body = functools.partial(_sc_agg_body, True)
    else:
        def body(x_hbm, src_hbm, dst_hbm, zagg_hbm, part_hbm, agg_sh,
                 src_v, dst_v, buf, sem):
            _sc_agg_body(False, x_hbm, src_hbm, dst_hbm, None, zagg_hbm,
                         None, part_hbm, None, agg_sh, None, src_v, dst_v,
                         None, buf, sem)
    return pl.kernel(body, out_type=tuple(out_type), mesh=_MESH,
                     scratch_types=tuple(scratch))


_sc_agg_deg = _make_sc_agg(True)
_sc_agg = _make_sc_agg(False)


def _tc_layer_body(final, x_ref, p_ref, degp_ref, W1_ref, b1_ref, g1_ref,
                   bt1_ref, W2_ref, b2_ref, g2_ref, bt2_ref, *out_refs):
    deg = degp_ref[0, :, 0:1] + degp_ref[1, :, 0:1]
    recip = 1.0 / jnp.maximum(deg, 1.0)
    h = x_ref[...] + (p_ref[0] + p_ref[1]) * recip
    t = jnp.dot(h, W1_ref[...], preferred_element_type=jnp.float32) + b1_ref[...]
    m = jnp.mean(t, axis=0)
    v = jnp.mean((t - m) ** 2, axis=0)
    t = jnp.maximum((t - m) * lax.rsqrt(v + 1e-5) * g1_ref[...] + bt1_ref[...], 0.0)
    t = jnp.dot(t, W2_ref[...], preferred_element_type=jnp.float32) + b2_ref[...]
    m = jnp.mean(t, axis=0)
    v = jnp.mean((t - m) ** 2, axis=0)
    t = jnp.maximum((t - m) * lax.rsqrt(v + 1e-5) * g2_ref[...] + bt2_ref[...], 0.0)
    out_refs[0][...] = t
    if final:
        out_refs[1][...] = jnp.mean(t, axis=0, keepdims=True)


def _make_tc_layer(final):
    outs = [jax.ShapeDtypeStruct((N, D), jnp.float32)]
    if final:
        outs.append(jax.ShapeDtypeStruct((1, D), jnp.float32))
    return pl.pallas_call(
        functools.partial(_tc_layer_body, final),
        out_shape=tuple(outs),
    )


_tc_layer = _make_tc_layer(False)
_tc_layer_final = _make_tc_layer(True)


def kernel(features, edge_index, l0_W1, l0_b1, l0_g1, l0_bt1, l0_W2, l0_b2,
           l0_g2, l0_bt2, l1_W1, l1_b1, l1_g1, l1_bt1, l1_W2, l1_b2, l1_g2,
           l1_bt2):
    src = edge_index[0]
    dst = edge_index[1]
    pad = E_PAD - E
    src_p = jnp.concatenate([src, jnp.zeros((pad,), jnp.int32)]).reshape(-1, CHUNK)
    dst_p = jnp.concatenate([dst, jnp.full((pad,), N, jnp.int32)]).reshape(-1, CHUNK)
    ones = jnp.ones((CHUNK, 16), jnp.float32)
    zagg = jnp.zeros((ZROWS, D), jnp.float32)
    zdeg = jnp.zeros((ZROWS, 16), jnp.float32)

    parts0, degp = _sc_agg_deg(features, src_p, dst_p, ones, zagg, zdeg)
    x1 = _tc_layer(features, parts0, degp, l0_W1, l0_b1, l0_g1, l0_bt1,
                   l0_W2, l0_b2, l0_g2, l0_bt2)[0]
    parts1 = _sc_agg(x1, src_p, dst_p, zagg)[0]
    x2, pool = _tc_layer_final(x1, parts1, degp, l1_W1, l1_b1, l1_g1, l1_bt1,
                               l1_W2, l1_b2, l1_g2, l1_bt2)
    return (pool, x2)


# speedup vs baseline: 2.8420x; 2.8420x over previous
import functools

import jax
import jax.numpy as jnp
from jax import lax
from jax.experimental import pallas as pl
from jax.experimental.pallas import tpu as pltpu
from jax.experimental.pallas import tpu_sc as plsc

N = 10000
D = 128
E = 320000

NC = 2
NS = 16
CHUNK = 128
CPT = 80
IDX_BLK = 8
CHUNKS_PER_CORE = NS * CPT
TOT_CHUNKS = NC * CHUNKS_PER_CORE
E_PAD = TOT_CHUNKS * CHUNK
NPAD = 10112
ZROWS = NPAD // NS
OROWS = ZROWS

_MESH = plsc.VectorSubcoreMesh(core_axis_name="c", subcore_axis_name="s")


def _sc_agg_body(with_deg, x_hbm, src_hbm, dst_hbm, ones_hbm, zagg_hbm,
                 part_hbm, degp_hbm, agg_sh, src_v, dst_v, buf, sem):
    c = lax.axis_index("c")
    s = lax.axis_index("s")
    base = c * CHUNKS_PER_CORE + s * CPT
    pltpu.sync_copy(zagg_hbm, agg_sh.at[pl.ds(s * ZROWS, ZROWS)])
    plsc.subcore_barrier()

    if with_deg:
        pltpu.sync_copy(ones_hbm, buf)

        def deg_outer(o, carry):
            pltpu.sync_copy(dst_hbm.at[pl.ds(base + o * IDX_BLK, IDX_BLK)],
                            dst_v)

            def deg_step(i, c2):
                pltpu.sync_copy(buf, agg_sh.at[dst_v.at[i]], add=True)
                return c2

            lax.fori_loop(0, IDX_BLK, deg_step, 0)
            return carry

        lax.fori_loop(0, CPT // IDX_BLK, deg_outer, 0)
        plsc.subcore_barrier()
        pltpu.sync_copy(agg_sh.at[pl.ds(s * OROWS, OROWS)],
                        degp_hbm.at[c, pl.ds(s * OROWS, OROWS)])
        pltpu.sync_copy(zagg_hbm, agg_sh.at[pl.ds(s * ZROWS, ZROWS)])
        plsc.subcore_barrier()

    def outer(o, carry):
        blk = base + o * IDX_BLK
        pltpu.sync_copy(src_hbm.at[pl.ds(blk, IDX_BLK)], src_v)
        pltpu.sync_copy(dst_hbm.at[pl.ds(blk, IDX_BLK)], dst_v)

        def step(i, c2):
            pltpu.async_copy(x_hbm.at[src_v.at[i]], buf, sem).wait()
            pltpu.sync_copy(buf, agg_sh.at[dst_v.at[i]], add=True)
            return c2

        lax.fori_loop(0, IDX_BLK, step, 0)
        return carry

    lax.fori_loop(0, CPT // IDX_BLK, outer, 0)
    plsc.subcore_barrier()
    pltpu.sync_copy(agg_sh.at[pl.ds(s * OROWS, OROWS)],
                    part_hbm.at[c, pl.ds(s * OROWS, OROWS)])


def _make_sc_agg(with_deg):
    out_type = [jax.ShapeDtypeStruct((NC, NPAD, D), jnp.float32)]
    if with_deg:
        out_type.append(jax.ShapeDtypeStruct((NC, NPAD, D), jnp.float32))
    scratch = (
        pltpu.VMEM_SHARED((NPAD, D), jnp.float32),
        pltpu.VMEM((IDX_BLK, CHUNK), jnp.int32),
        pltpu.VMEM((IDX_BLK, CHUNK), jnp.int32),
        pltpu.VMEM((CHUNK, D), jnp.float32),
        pltpu.SemaphoreType.DMA,
    )

    if with_deg:
        body = functools.partial(_sc_agg_body, True)
    else:
        def body(x_hbm, src_hbm, dst_hbm, zagg_hbm, part_hbm, agg_sh,
                 src_v, dst_v, buf, sem):
            _sc_agg_body(False, x_hbm, src_hbm, dst_hbm, None, zagg_hbm,
                         part_hbm, None, agg_sh, src_v, dst_v, buf, sem)
    return pl.kernel(body, out_type=tuple(out_type), mesh=_MESH,
                     scratch_types=scratch)


_sc_agg_deg = _make_sc_agg(True)
_sc_agg = _make_sc_agg(False)


def _tc_layer_body(final, x_ref, p_ref, degp_ref, W1_ref, b1_ref, g1_ref,
                   bt1_ref, W2_ref, b2_ref, g2_ref, bt2_ref, *out_refs):
    deg = degp_ref[0, :N, 0:1] + degp_ref[1, :N, 0:1]
    recip = 1.0 / jnp.maximum(deg, 1.0)
    h = x_ref[...] + (p_ref[0, :N] + p_ref[1, :N]) * recip
    t = jnp.dot(h, W1_ref[...], preferred_element_type=jnp.float32) + b1_ref[...]
    m = jnp.mean(t, axis=0)
    v = jnp.mean((t - m) ** 2, axis=0)
    t = jnp.maximum((t - m) * lax.rsqrt(v + 1e-5) * g1_ref[...] + bt1_ref[...], 0.0)
    t = jnp.dot(t, W2_ref[...], preferred_element_type=jnp.float32) + b2_ref[...]
    m = jnp.mean(t, axis=0)
    v = jnp.mean((t - m) ** 2, axis=0)
    t = jnp.maximum((t - m) * lax.rsqrt(v + 1e-5) * g2_ref[...] + bt2_ref[...], 0.0)
    out_refs[0][...] = t
    if final:
        out_refs[1][...] = jnp.mean(t, axis=0, keepdims=True)


def _make_tc_layer(final):
    outs = [jax.ShapeDtypeStruct((N, D), jnp.float32)]
    if final:
        outs.append(jax.ShapeDtypeStruct((1, D), jnp.float32))
    return pl.pallas_call(
        functools.partial(_tc_layer_body, final),
        out_shape=tuple(outs),
    )


_tc_layer = _make_tc_layer(False)
_tc_layer_final = _make_tc_layer(True)


def kernel(features, edge_index, l0_W1, l0_b1, l0_g1, l0_bt1, l0_W2, l0_b2,
           l0_g2, l0_bt2, l1_W1, l1_b1, l1_g1, l1_bt1, l1_W2, l1_b2, l1_g2,
           l1_bt2):
    src = edge_index[0]
    dst = edge_index[1]
    pad = E_PAD - E
    src_p = jnp.concatenate([src, jnp.zeros((pad,), jnp.int32)]).reshape(-1, CHUNK)
    dst_p = jnp.concatenate([dst, jnp.full((pad,), N, jnp.int32)]).reshape(-1, CHUNK)
    ones = jnp.ones((CHUNK, D), jnp.float32)
    zagg = jnp.zeros((ZROWS, D), jnp.float32)

    parts0, degp = _sc_agg_deg(features, src_p, dst_p, ones, zagg)
    x1 = _tc_layer(features, parts0, degp, l0_W1, l0_b1, l0_g1, l0_bt1,
                   l0_W2, l0_b2, l0_g2, l0_bt2)[0]
    parts1 = _sc_agg(x1, src_p, dst_p, zagg)[0]
    x2, pool = _tc_layer_final(x1, parts1, degp, l1_W1, l1_b1, l1_g1, l1_bt1,
                               l1_W2, l1_b2, l1_g2, l1_bt2)
    return (pool, x2)


# speedup vs baseline: 3.0778x; 1.0830x over previous
import functools

import jax
import jax.numpy as jnp
from jax import lax
from jax.experimental import pallas as pl
from jax.experimental.pallas import tpu as pltpu
from jax.experimental.pallas import tpu_sc as plsc

N = 10000
D = 128
E = 320000

NC = 2
NS = 16
CHUNK = 128
CPT = 80
IDX_BLK = 8
CHUNKS_PER_CORE = NS * CPT
TOT_CHUNKS = NC * CHUNKS_PER_CORE
E_PAD = TOT_CHUNKS * CHUNK
NPAD = 10112
ZROWS = NPAD // NS
OROWS = ZROWS

_MESH = plsc.VectorSubcoreMesh(core_axis_name="c", subcore_axis_name="s")


def _sc_agg_body(with_deg, x_hbm, src_hbm, dst_hbm, ones_hbm, zagg_hbm,
                 part_hbm, degp_hbm, agg_sh, src_v, dst_v, buf_a, buf_b,
                 gsem_a, gsem_b, ssem_a, ssem_b):
    c = lax.axis_index("c")
    s = lax.axis_index("s")
    base = c * CHUNKS_PER_CORE + s * CPT
    bufs = (buf_a, buf_b)
    gsems = (gsem_a, gsem_b)
    ssems = (ssem_a, ssem_b)
    pltpu.sync_copy(zagg_hbm, agg_sh.at[pl.ds(s * ZROWS, ZROWS)])
    plsc.subcore_barrier()

    if with_deg:
        pltpu.sync_copy(ones_hbm, buf_a)

        def deg_outer(o, carry):
            pltpu.sync_copy(dst_hbm.at[pl.ds(base + o * IDX_BLK, IDX_BLK)],
                            dst_v)
            for i in range(IDX_BLK):
                pltpu.async_copy(buf_a, agg_sh.at[dst_v.at[i]], ssem_a,
                                 add=True)
            for i in range(IDX_BLK):
                pltpu.make_async_copy(buf_a, agg_sh.at[dst_v.at[i]],
                                      ssem_a).wait()
            return carry

        lax.fori_loop(0, CPT // IDX_BLK, deg_outer, 0)
        plsc.subcore_barrier()
        pltpu.sync_copy(agg_sh.at[pl.ds(s * OROWS, OROWS)],
                        degp_hbm.at[c, pl.ds(s * OROWS, OROWS)])
        pltpu.sync_copy(zagg_hbm, agg_sh.at[pl.ds(s * ZROWS, ZROWS)])
        plsc.subcore_barrier()

    def outer(o, carry):
        blk = base + o * IDX_BLK
        pltpu.sync_copy(src_hbm.at[pl.ds(blk, IDX_BLK)], src_v)
        pltpu.sync_copy(dst_hbm.at[pl.ds(blk, IDX_BLK)], dst_v)
        pltpu.async_copy(x_hbm.at[src_v.at[0]], buf_a, gsem_a)
        pltpu.async_copy(x_hbm.at[src_v.at[1]], buf_b, gsem_b)
        for i in range(IDX_BLK):
            b = bufs[i % 2]
            pltpu.make_async_copy(x_hbm.at[src_v.at[i]], b, gsems[i % 2]).wait()
            pltpu.async_copy(b, agg_sh.at[dst_v.at[i]], ssems[i % 2], add=True)
            if i + 2 < IDX_BLK:
                pltpu.make_async_copy(b, agg_sh.at[dst_v.at[i]],
                                      ssems[i % 2]).wait()
                pltpu.async_copy(x_hbm.at[src_v.at[i + 2]], b, gsems[i % 2])
        for i in (IDX_BLK - 2, IDX_BLK - 1):
            pltpu.make_async_copy(bufs[i % 2], agg_sh.at[dst_v.at[i]],
                                  ssems[i % 2]).wait()
        return carry

    lax.fori_loop(0, CPT // IDX_BLK, outer, 0)
    plsc.subcore_barrier()
    pltpu.sync_copy(agg_sh.at[pl.ds(s * OROWS, OROWS)],
                    part_hbm.at[c, pl.ds(s * OROWS, OROWS)])


def _make_sc_agg(with_deg):
    out_type = [jax.ShapeDtypeStruct((NC, NPAD, D), jnp.float32)]
    if with_deg:
        out_type.append(jax.ShapeDtypeStruct((NC, NPAD, D), jnp.float32))
    scratch = (
        pltpu.VMEM_SHARED((NPAD, D), jnp.float32),
        pltpu.VMEM((IDX_BLK, CHUNK), jnp.int32),
        pltpu.VMEM((IDX_BLK, CHUNK), jnp.int32),
        pltpu.VMEM((CHUNK, D), jnp.float32),
        pltpu.VMEM((CHUNK, D), jnp.float32),
        pltpu.SemaphoreType.DMA,
        pltpu.SemaphoreType.DMA,
        pltpu.SemaphoreType.DMA,
        pltpu.SemaphoreType.DMA,
    )

    if with_deg:
        body = functools.partial(_sc_agg_body, True)
    else:
        def body(x_hbm, src_hbm, dst_hbm, zagg_hbm, part_hbm, agg_sh,
                 src_v, dst_v, buf_a, buf_b, gsem_a, gsem_b, ssem_a, ssem_b):
            _sc_agg_body(False, x_hbm, src_hbm, dst_hbm, None, zagg_hbm,
                         part_hbm, None, agg_sh, src_v, dst_v, buf_a, buf_b,
                         gsem_a, gsem_b, ssem_a, ssem_b)
    return pl.kernel(body, out_type=tuple(out_type), mesh=_MESH,
                     scratch_types=scratch)


_sc_agg_deg = _make_sc_agg(True)
_sc_agg = _make_sc_agg(False)


def _tc_layer_body(final, x_ref, p_ref, degp_ref, W1_ref, b1_ref, g1_ref,
                   bt1_ref, W2_ref, b2_ref, g2_ref, bt2_ref, *out_refs):
    deg = degp_ref[0, :N, 0:1] + degp_ref[1, :N, 0:1]
    recip = 1.0 / jnp.maximum(deg, 1.0)
    h = x_ref[...] + (p_ref[0, :N] + p_ref[1, :N]) * recip
    t = jnp.dot(h, W1_ref[...], preferred_element_type=jnp.float32) + b1_ref[...]
    m = jnp.mean(t, axis=0)
    v = jnp.mean((t - m) ** 2, axis=0)
    t = jnp.maximum((t - m) * lax.rsqrt(v + 1e-5) * g1_ref[...] + bt1_ref[...], 0.0)
    t = jnp.dot(t, W2_ref[...], preferred_element_type=jnp.float32) + b2_ref[...]
    m = jnp.mean(t, axis=0)
    v = jnp.mean((t - m) ** 2, axis=0)
    t = jnp.maximum((t - m) * lax.rsqrt(v + 1e-5) * g2_ref[...] + bt2_ref[...], 0.0)
    out_refs[0][...] = t
    if final:
        out_refs[1][...] = jnp.mean(t, axis=0, keepdims=True)


def _make_tc_layer(final):
    outs = [jax.ShapeDtypeStruct((N, D), jnp.float32)]
    if final:
        outs.append(jax.ShapeDtypeStruct((1, D), jnp.float32))
    return pl.pallas_call(
        functools.partial(_tc_layer_body, final),
        out_shape=tuple(outs),
    )


_tc_layer = _make_tc_layer(False)
_tc_layer_final = _make_tc_layer(True)


def kernel(features, edge_index, l0_W1, l0_b1, l0_g1, l0_bt1, l0_W2, l0_b2,
           l0_g2, l0_bt2, l1_W1, l1_b1, l1_g1, l1_bt1, l1_W2, l1_b2, l1_g2,
           l1_bt2):
    src = edge_index[0]
    dst = edge_index[1]
    pad = E_PAD - E
    src_p = jnp.concatenate([src, jnp.zeros((pad,), jnp.int32)]).reshape(-1, CHUNK)
    dst_p = jnp.concatenate([dst, jnp.full((pad,), N, jnp.int32)]).reshape(-1, CHUNK)
    ones = jnp.ones((CHUNK, D), jnp.float32)
    zagg = jnp.zeros((ZROWS, D), jnp.float32)

    parts0, degp = _sc_agg_deg(features, src_p, dst_p, ones, zagg)
    x1 = _tc_layer(features, parts0, degp, l0_W1, l0_b1, l0_g1, l0_bt1,
                   l0_W2, l0_b2, l0_g2, l0_bt2)[0]
    parts1 = _sc_agg(x1, src_p, dst_p, zagg)[0]
    x2, pool = _tc_layer_final(x1, parts1, degp, l1_W1, l1_b1, l1_g1, l1_bt1,
                               l1_W2, l1_b2, l1_g2, l1_bt2)
    return (pool, x2)


# speedup vs baseline: 3.0779x; 1.0000x over previous
import functools

import jax
import jax.numpy as jnp
from jax import lax
from jax.experimental import pallas as pl
from jax.experimental.pallas import tpu as pltpu
from jax.experimental.pallas import tpu_sc as plsc

N = 10000
D = 128
E = 320000

NC = 2
NS = 16
CHUNK = 128
CPT = 80
IDX_BLK = 8
CHUNKS_PER_CORE = NS * CPT
TOT_CHUNKS = NC * CHUNKS_PER_CORE
E_PAD = TOT_CHUNKS * CHUNK
NPAD = 10112
ZROWS = NPAD // NS
OROWS = ZROWS

_MESH = plsc.VectorSubcoreMesh(core_axis_name="c", subcore_axis_name="s")


def _sc_agg_body(with_deg, x_hbm, src_hbm, dst_hbm, ones_hbm, zagg_hbm,
                 part_hbm, degp_hbm, agg_sh, src_v, dst_v, buf_a, buf_b,
                 gsem_a, gsem_b, ssem_a, ssem_b):
    c = lax.axis_index("c")
    s = lax.axis_index("s")
    base = c * CHUNKS_PER_CORE + s * CPT
    bufs = (buf_a, buf_b)
    gsems = (gsem_a, gsem_b)
    ssems = (ssem_a, ssem_b)
    pltpu.sync_copy(zagg_hbm, agg_sh.at[pl.ds(s * ZROWS, ZROWS)])
    plsc.subcore_barrier()

    if with_deg:
        pltpu.sync_copy(ones_hbm, buf_a)

        def deg_outer(o, carry):
            pltpu.sync_copy(dst_hbm.at[pl.ds(base + o * IDX_BLK, IDX_BLK)],
                            dst_v)
            for i in range(IDX_BLK):
                pltpu.async_copy(buf_a, agg_sh.at[dst_v.at[i]], ssem_a,
                                 add=True)
            for i in range(IDX_BLK):
                pltpu.make_async_copy(buf_a, agg_sh.at[dst_v.at[i]],
                                      ssem_a).wait()
            return carry

        lax.fori_loop(0, CPT // IDX_BLK, deg_outer, 0)
        plsc.subcore_barrier()
        pltpu.sync_copy(agg_sh.at[pl.ds(s * OROWS, OROWS)],
                        degp_hbm.at[c, pl.ds(s * OROWS, OROWS)])
        pltpu.sync_copy(zagg_hbm, agg_sh.at[pl.ds(s * ZROWS, ZROWS)])
        plsc.subcore_barrier()

    half = CHUNK // 2

    def fire_gather_pair(i, b, gsem):
        pltpu.async_copy(x_hbm.at[src_v.at[i, pl.ds(0, half)]],
                         b.at[pl.ds(0, half)], gsem)
        pltpu.async_copy(x_hbm.at[src_v.at[i, pl.ds(half, half)]],
                         b.at[pl.ds(half, half)], gsem)

    def wait_gather_pair(i, b, gsem):
        pltpu.make_async_copy(x_hbm.at[src_v.at[i, pl.ds(0, half)]],
                              b.at[pl.ds(0, half)], gsem).wait()
        pltpu.make_async_copy(x_hbm.at[src_v.at[i, pl.ds(half, half)]],
                              b.at[pl.ds(half, half)], gsem).wait()

    def outer(o, carry):
        blk = base + o * IDX_BLK
        pltpu.sync_copy(src_hbm.at[pl.ds(blk, IDX_BLK)], src_v)
        pltpu.sync_copy(dst_hbm.at[pl.ds(blk, IDX_BLK)], dst_v)
        fire_gather_pair(0, buf_a, gsem_a)
        fire_gather_pair(1, buf_b, gsem_b)
        for i in range(IDX_BLK):
            b = bufs[i % 2]
            wait_gather_pair(i, b, gsems[i % 2])
            pltpu.async_copy(b, agg_sh.at[dst_v.at[i]], ssems[i % 2], add=True)
            if i + 2 < IDX_BLK:
                pltpu.make_async_copy(b, agg_sh.at[dst_v.at[i]],
                                      ssems[i % 2]).wait()
                fire_gather_pair(i + 2, b, gsems[i % 2])
        for i in (IDX_BLK - 2, IDX_BLK - 1):
            pltpu.make_async_copy(bufs[i % 2], agg_sh.at[dst_v.at[i]],
                                  ssems[i % 2]).wait()
        return carry

    lax.fori_loop(0, CPT // IDX_BLK, outer, 0)
    plsc.subcore_barrier()
    pltpu.sync_copy(agg_sh.at[pl.ds(s * OROWS, OROWS)],
                    part_hbm.at[c, pl.ds(s * OROWS, OROWS)])


def _make_sc_agg(with_deg):
    out_type = [jax.ShapeDtypeStruct((NC, NPAD, D), jnp.float32)]
    if with_deg:
        out_type.append(jax.ShapeDtypeStruct((NC, NPAD, D), jnp.float32))
    scratch = (
        pltpu.VMEM_SHARED((NPAD, D), jnp.float32),
        pltpu.VMEM((IDX_BLK, CHUNK), jnp.int32),
        pltpu.VMEM((IDX_BLK, CHUNK), jnp.int32),
        pltpu.VMEM((CHUNK, D), jnp.float32),
        pltpu.VMEM((CHUNK, D), jnp.float32),
        pltpu.SemaphoreType.DMA,
        pltpu.SemaphoreType.DMA,
        pltpu.SemaphoreType.DMA,
        pltpu.SemaphoreType.DMA,
    )

    if with_deg:
        body = functools.partial(_sc_agg_body, True)
    else:
        def body(x_hbm, src_hbm, dst_hbm, zagg_hbm, part_hbm, agg_sh,
                 src_v, dst_v, buf_a, buf_b, gsem_a, gsem_b, ssem_a, ssem_b):
            _sc_agg_body(False, x_hbm, src_hbm, dst_hbm, None, zagg_hbm,
                         part_hbm, None, agg_sh, src_v, dst_v, buf_a, buf_b,
                         gsem_a, gsem_b, ssem_a, ssem_b)
    return pl.kernel(body, out_type=tuple(out_type), mesh=_MESH,
                     scratch_types=scratch)


_sc_agg_deg = _make_sc_agg(True)
_sc_agg = _make_sc_agg(False)


def _tc_layer_body(final, x_ref, p_ref, degp_ref, W1_ref, b1_ref, g1_ref,
                   bt1_ref, W2_ref, b2_ref, g2_ref, bt2_ref, *out_refs):
    deg = degp_ref[0, :N, 0:1] + degp_ref[1, :N, 0:1]
    recip = 1.0 / jnp.maximum(deg, 1.0)
    h = x_ref[...] + (p_ref[0, :N] + p_ref[1, :N]) * recip
    t = jnp.dot(h, W1_ref[...], preferred_element_type=jnp.float32) + b1_ref[...]
    m = jnp.mean(t, axis=0)
    v = jnp.mean((t - m) ** 2, axis=0)
    t = jnp.maximum((t - m) * lax.rsqrt(v + 1e-5) * g1_ref[...] + bt1_ref[...], 0.0)
    t = jnp.dot(t, W2_ref[...], preferred_element_type=jnp.float32) + b2_ref[...]
    m = jnp.mean(t, axis=0)
    v = jnp.mean((t - m) ** 2, axis=0)
    t = jnp.maximum((t - m) * lax.rsqrt(v + 1e-5) * g2_ref[...] + bt2_ref[...], 0.0)
    out_refs[0][...] = t
    if final:
        out_refs[1][...] = jnp.mean(t, axis=0, keepdims=True)


def _make_tc_layer(final):
    outs = [jax.ShapeDtypeStruct((N, D), jnp.float32)]
    if final:
        outs.append(jax.ShapeDtypeStruct((1, D), jnp.float32))
    return pl.pallas_call(
        functools.partial(_tc_layer_body, final),
        out_shape=tuple(outs),
    )


_tc_layer = _make_tc_layer(False)
_tc_layer_final = _make_tc_layer(True)


def kernel(features, edge_index, l0_W1, l0_b1, l0_g1, l0_bt1, l0_W2, l0_b2,
           l0_g2, l0_bt2, l1_W1, l1_b1, l1_g1, l1_bt1, l1_W2, l1_b2, l1_g2,
           l1_bt2):
    src = edge_index[0]
    dst = edge_index[1]
    pad = E_PAD - E
    src_p = jnp.concatenate([src, jnp.zeros((pad,), jnp.int32)]).reshape(-1, CHUNK)
    dst_p = jnp.concatenate([dst, jnp.full((pad,), N, jnp.int32)]).reshape(-1, CHUNK)
    ones = jnp.ones((CHUNK, D), jnp.float32)
    zagg = jnp.zeros((ZROWS, D), jnp.float32)

    parts0, degp = _sc_agg_deg(features, src_p, dst_p, ones, zagg)
    x1 = _tc_layer(features, parts0, degp, l0_W1, l0_b1, l0_g1, l0_bt1,
                   l0_W2, l0_b2, l0_g2, l0_bt2)[0]
    parts1 = _sc_agg(x1, src_p, dst_p, zagg)[0]
    x2, pool = _tc_layer_final(x1, parts1, degp, l1_W1, l1_b1, l1_g1, l1_bt1,
                               l1_W2, l1_b2, l1_g2, l1_bt2)
    return (pool, x2)


# speedup vs baseline: 3.6790x; 1.1953x over previous
import functools

import jax
import jax.numpy as jnp
from jax import lax
from jax.experimental import pallas as pl
from jax.experimental.pallas import tpu as pltpu
from jax.experimental.pallas import tpu_sc as plsc

N = 10000
D = 128
E = 320000

NC = 2
NS = 16
CHUNK = 128
CPT = 80
IDX_BLK = 8
CHUNKS_PER_CORE = NS * CPT
TOT_CHUNKS = NC * CHUNKS_PER_CORE
E_PAD = TOT_CHUNKS * CHUNK
NPAD = 10112
ZROWS = NPAD // NS
OROWS = ZROWS

_MESH = plsc.VectorSubcoreMesh(core_axis_name="c", subcore_axis_name="s")


def _sc_agg_body(with_deg, x_hbm, src_hbm, dst_hbm, ones_hbm, zagg_hbm,
                 part_hbm, degp_hbm, agg_sh, src_v, dst_v, buf_a, buf_b,
                 gsem_a, gsem_b, ssem_a, ssem_b):
    c = lax.axis_index("c")
    s = lax.axis_index("s")
    base = c * CHUNKS_PER_CORE + s * CPT
    bufs = (buf_a, buf_b)
    gsems = (gsem_a, gsem_b)
    ssems = (ssem_a, ssem_b)
    pltpu.sync_copy(zagg_hbm, agg_sh.at[pl.ds(s * ZROWS, ZROWS)])
    plsc.subcore_barrier()

    if with_deg:
        pltpu.sync_copy(ones_hbm, buf_a)

        def deg_outer(o, carry):
            pltpu.sync_copy(dst_hbm.at[pl.ds(base + o * IDX_BLK, IDX_BLK)],
                            dst_v)
            for i in range(IDX_BLK):
                pltpu.async_copy(buf_a, agg_sh.at[dst_v.at[i]], ssem_a,
                                 add=True)
            for i in range(IDX_BLK):
                pltpu.make_async_copy(buf_a, agg_sh.at[dst_v.at[i]],
                                      ssem_a).wait()
            return carry

        lax.fori_loop(0, CPT // IDX_BLK, deg_outer, 0)
        plsc.subcore_barrier()
        pltpu.sync_copy(agg_sh.at[pl.ds(s * OROWS, OROWS)],
                        degp_hbm.at[c, pl.ds(s * OROWS, OROWS)])
        pltpu.sync_copy(zagg_hbm, agg_sh.at[pl.ds(s * ZROWS, ZROWS)])
        plsc.subcore_barrier()

    half = CHUNK // 2

    def fire_gather_pair(i, b, gsem):
        pltpu.async_copy(x_hbm.at[src_v.at[i, pl.ds(0, half)]],
                         b.at[pl.ds(0, half)], gsem)
        pltpu.async_copy(x_hbm.at[src_v.at[i, pl.ds(half, half)]],
                         b.at[pl.ds(half, half)], gsem)

    def wait_gather_pair(i, b, gsem):
        pltpu.make_async_copy(x_hbm.at[src_v.at[i, pl.ds(0, half)]],
                              b.at[pl.ds(0, half)], gsem).wait()
        pltpu.make_async_copy(x_hbm.at[src_v.at[i, pl.ds(half, half)]],
                              b.at[pl.ds(half, half)], gsem).wait()

    def outer(o, carry):
        blk = base + o * IDX_BLK
        pltpu.sync_copy(src_hbm.at[pl.ds(blk, IDX_BLK)], src_v)
        pltpu.sync_copy(dst_hbm.at[pl.ds(blk, IDX_BLK)], dst_v)
        fire_gather_pair(0, buf_a, gsem_a)
        fire_gather_pair(1, buf_b, gsem_b)
        for i in range(IDX_BLK):
            b = bufs[i % 2]
            wait_gather_pair(i, b, gsems[i % 2])
            pltpu.async_copy(b, agg_sh.at[dst_v.at[i]], ssems[i % 2], add=True)
            if i + 2 < IDX_BLK:
                pltpu.make_async_copy(b, agg_sh.at[dst_v.at[i]],
                                      ssems[i % 2]).wait()
                fire_gather_pair(i + 2, b, gsems[i % 2])
        for i in (IDX_BLK - 2, IDX_BLK - 1):
            pltpu.make_async_copy(bufs[i % 2], agg_sh.at[dst_v.at[i]],
                                  ssems[i % 2]).wait()
        return carry

    lax.fori_loop(0, CPT // IDX_BLK, outer, 0)
    plsc.subcore_barrier()
    pltpu.sync_copy(agg_sh.at[pl.ds(s * OROWS, OROWS)],
                    part_hbm.at[c, pl.ds(s * OROWS, OROWS)])


def _make_sc_agg(with_deg):
    out_type = [jax.ShapeDtypeStruct((NC, NPAD, D), jnp.float32)]
    if with_deg:
        out_type.append(jax.ShapeDtypeStruct((NC, NPAD, D), jnp.float32))
    scratch = (
        pltpu.VMEM_SHARED((NPAD, D), jnp.float32),
        pltpu.VMEM((IDX_BLK, CHUNK), jnp.int32),
        pltpu.VMEM((IDX_BLK, CHUNK), jnp.int32),
        pltpu.VMEM((CHUNK, D), jnp.float32),
        pltpu.VMEM((CHUNK, D), jnp.float32),
        pltpu.SemaphoreType.DMA,
        pltpu.SemaphoreType.DMA,
        pltpu.SemaphoreType.DMA,
        pltpu.SemaphoreType.DMA,
    )

    if with_deg:
        body = functools.partial(_sc_agg_body, True)
    else:
        def body(x_hbm, src_hbm, dst_hbm, zagg_hbm, part_hbm, agg_sh,
                 src_v, dst_v, buf_a, buf_b, gsem_a, gsem_b, ssem_a, ssem_b):
            _sc_agg_body(False, x_hbm, src_hbm, dst_hbm, None, zagg_hbm,
                         part_hbm, None, agg_sh, src_v, dst_v, buf_a, buf_b,
                         gsem_a, gsem_b, ssem_a, ssem_b)
    return pl.kernel(body, out_type=tuple(out_type), mesh=_MESH,
                     scratch_types=scratch)


_sc_agg_deg = _make_sc_agg(True)
_sc_agg = _make_sc_agg(False)


def _tc_layer_body(final, x_ref, p_ref, degp_ref, W1_ref, b1_ref, g1_ref,
                   bt1_ref, W2_ref, b2_ref, g2_ref, bt2_ref, *out_refs):
    deg = degp_ref[0, :N, 0:1] + degp_ref[1, :N, 0:1]
    recip = 1.0 / jnp.maximum(deg, 1.0)
    h = x_ref[...] + (p_ref[0, :N] + p_ref[1, :N]) * recip
    t = jnp.dot(h, W1_ref[...], preferred_element_type=jnp.float32) + b1_ref[...]
    m = jnp.mean(t, axis=0)
    v = jnp.mean((t - m) ** 2, axis=0)
    t = jnp.maximum((t - m) * lax.rsqrt(v + 1e-5) * g1_ref[...] + bt1_ref[...], 0.0)
    t = jnp.dot(t, W2_ref[...], preferred_element_type=jnp.float32) + b2_ref[...]
    m = jnp.mean(t, axis=0)
    v = jnp.mean((t - m) ** 2, axis=0)
    t = jnp.maximum((t - m) * lax.rsqrt(v + 1e-5) * g2_ref[...] + bt2_ref[...], 0.0)
    out_refs[0][...] = t
    if final:
        out_refs[1][...] = jnp.mean(t, axis=0, keepdims=True)


def _make_tc_layer(final):
    outs = [jax.ShapeDtypeStruct((N, D), jnp.float32)]
    if final:
        outs.append(jax.ShapeDtypeStruct((1, D), jnp.float32))
    return pl.pallas_call(
        functools.partial(_tc_layer_body, final),
        out_shape=tuple(outs),
    )


_tc_layer = _make_tc_layer(False)
_tc_layer_final = _make_tc_layer(True)


def kernel(features, edge_index, l0_W1, l0_b1, l0_g1, l0_bt1, l0_W2, l0_b2,
           l0_g2, l0_bt2, l1_W1, l1_b1, l1_g1, l1_bt1, l1_W2, l1_b2, l1_g2,
           l1_bt2):
    src = edge_index[0]
    dst = edge_index[1]
    half_e = E // 2
    pad_half = E_PAD // 2 - half_e
    zpad = jnp.zeros((pad_half,), jnp.int32)
    tpad = N + jnp.arange(pad_half, dtype=jnp.int32) % (NPAD - N)
    src_p = jnp.concatenate([src[:half_e], zpad, src[half_e:], zpad]).reshape(-1, CHUNK)
    dst_p = jnp.concatenate([dst[:half_e], tpad, dst[half_e:], tpad]).reshape(-1, CHUNK)
    ones = jnp.ones((CHUNK, D), jnp.float32)
    zagg = jnp.zeros((ZROWS, D), jnp.float32)

    parts0, degp = _sc_agg_deg(features, src_p, dst_p, ones, zagg)
    x1 = _tc_layer(features, parts0, degp, l0_W1, l0_b1, l0_g1, l0_bt1,
                   l0_W2, l0_b2, l0_g2, l0_bt2)[0]
    parts1 = _sc_agg(x1, src_p, dst_p, zagg)[0]
    x2, pool = _tc_layer_final(x1, parts1, degp, l1_W1, l1_b1, l1_g1, l1_bt1,
                               l1_W2, l1_b2, l1_g2, l1_bt2)
    return (pool, x2)


# speedup vs baseline: 3.7536x; 1.0203x over previous
import functools

import jax
import jax.numpy as jnp
from jax import lax
from jax.experimental import pallas as pl
from jax.experimental.pallas import tpu as pltpu
from jax.experimental.pallas import tpu_sc as plsc

N = 10000
D = 128
E = 320000

NC = 2
NS = 16
CHUNK = 128
CPT = 80
IDX_BLK = 8
CHUNKS_PER_CORE = NS * CPT
TOT_CHUNKS = NC * CHUNKS_PER_CORE
E_PAD = TOT_CHUNKS * CHUNK
NPAD = 10112
ZROWS = NPAD // NS
OROWS = ZROWS

_MESH = plsc.VectorSubcoreMesh(core_axis_name="c", subcore_axis_name="s")


def _sc_agg_body(with_deg, x_hbm, src_hbm, dst_hbm, ones_hbm, zagg_hbm,
                 part_hbm, degp_hbm, agg_sh, src_v0, src_v1, dst_v0, dst_v1,
                 buf_a, buf_b, gsem_a, gsem_b, ssem_a, ssem_b, isem0, isem1):
    c = lax.axis_index("c")
    s = lax.axis_index("s")
    base = c * CHUNKS_PER_CORE + s * CPT
    bufs = (buf_a, buf_b)
    gsems = (gsem_a, gsem_b)
    ssems = (ssem_a, ssem_b)
    src_slots = (src_v0, src_v1)
    dst_slots = (dst_v0, dst_v1)
    isems = (isem0, isem1)
    NHALF = CPT // IDX_BLK // 2

    pltpu.sync_copy(zagg_hbm, agg_sh.at[pl.ds(s * ZROWS, ZROWS)])
    plsc.subcore_barrier()

    def prefetch(b, slot, with_src):
        blk = base + b * IDX_BLK
        if with_src:
            pltpu.async_copy(src_hbm.at[pl.ds(blk, IDX_BLK)],
                             src_slots[slot], isems[slot])
        pltpu.async_copy(dst_hbm.at[pl.ds(blk, IDX_BLK)],
                         dst_slots[slot], isems[slot])

    def wait_prefetch(b, slot, with_src):
        blk = base + b * IDX_BLK
        if with_src:
            pltpu.make_async_copy(src_hbm.at[pl.ds(blk, IDX_BLK)],
                                  src_slots[slot], isems[slot]).wait()
        pltpu.make_async_copy(dst_hbm.at[pl.ds(blk, IDX_BLK)],
                              dst_slots[slot], isems[slot]).wait()

    def run_pass(process_block, with_src):
        prefetch(0, 0, with_src)
        wait_prefetch(0, 0, with_src)
        prefetch(1, 1, with_src)

        def outer2(o2, carry):
            b0 = 2 * o2
            process_block(0)
            wait_prefetch(b0 + 1, 1, with_src)

            @pl.when(o2 < NHALF - 1)
            def _():
                prefetch(b0 + 2, 0, with_src)

            process_block(1)

            @pl.when(o2 < NHALF - 1)
            def _():
                prefetch(b0 + 3, 1, with_src)
                wait_prefetch(b0 + 2, 0, with_src)

            return carry

        lax.fori_loop(0, NHALF, outer2, 0)

    if with_deg:
        pltpu.sync_copy(ones_hbm, buf_a)

        def deg_block(slot):
            dv = dst_slots[slot]
            for i in range(IDX_BLK):
                pltpu.async_copy(buf_a, agg_sh.at[dv.at[i]], ssem_a,
                                 add=True)
            for i in range(IDX_BLK):
                pltpu.make_async_copy(buf_a, agg_sh.at[dv.at[i]],
                                      ssem_a).wait()

        run_pass(deg_block, with_src=False)
        plsc.subcore_barrier()
        pltpu.sync_copy(agg_sh.at[pl.ds(s * OROWS, OROWS)],
                        degp_hbm.at[c, pl.ds(s * OROWS, OROWS)])
        pltpu.sync_copy(zagg_hbm, agg_sh.at[pl.ds(s * ZROWS, ZROWS)])
        plsc.subcore_barrier()

    def main_block(slot):
        sv = src_slots[slot]
        dv = dst_slots[slot]
        pltpu.async_copy(x_hbm.at[sv.at[0]], buf_a, gsem_a)
        pltpu.async_copy(x_hbm.at[sv.at[1]], buf_b, gsem_b)
        for i in range(IDX_BLK):
            b = bufs[i % 2]
            pltpu.make_async_copy(x_hbm.at[sv.at[i]], b, gsems[i % 2]).wait()
            pltpu.async_copy(b, agg_sh.at[dv.at[i]], ssems[i % 2], add=True)
            if i + 2 < IDX_BLK:
                pltpu.make_async_copy(b, agg_sh.at[dv.at[i]],
                                      ssems[i % 2]).wait()
                pltpu.async_copy(x_hbm.at[sv.at[i + 2]], b, gsems[i % 2])
        for i in (IDX_BLK - 2, IDX_BLK - 1):
            pltpu.make_async_copy(bufs[i % 2], agg_sh.at[dv.at[i]],
                                  ssems[i % 2]).wait()

    run_pass(main_block, with_src=True)
    plsc.subcore_barrier()
    pltpu.sync_copy(agg_sh.at[pl.ds(s * OROWS, OROWS)],
                    part_hbm.at[c, pl.ds(s * OROWS, OROWS)])


def _make_sc_agg(with_deg):
    out_type = [jax.ShapeDtypeStruct((NC, NPAD, D), jnp.float32)]
    if with_deg:
        out_type.append(jax.ShapeDtypeStruct((NC, NPAD, D), jnp.float32))
    scratch = (
        pltpu.VMEM_SHARED((NPAD, D), jnp.float32),
        pltpu.VMEM((IDX_BLK, CHUNK), jnp.int32),
        pltpu.VMEM((IDX_BLK, CHUNK), jnp.int32),
        pltpu.VMEM((IDX_BLK, CHUNK), jnp.int32),
        pltpu.VMEM((IDX_BLK, CHUNK), jnp.int32),
        pltpu.VMEM((CHUNK, D), jnp.float32),
        pltpu.VMEM((CHUNK, D), jnp.float32),
        pltpu.SemaphoreType.DMA,
        pltpu.SemaphoreType.DMA,
        pltpu.SemaphoreType.DMA,
        pltpu.SemaphoreType.DMA,
        pltpu.SemaphoreType.DMA,
        pltpu.SemaphoreType.DMA,
    )

    if with_deg:
        body = functools.partial(_sc_agg_body, True)
    else:
        def body(x_hbm, src_hbm, dst_hbm, zagg_hbm, part_hbm, agg_sh,
                 src_v0, src_v1, dst_v0, dst_v1, buf_a, buf_b,
                 gsem_a, gsem_b, ssem_a, ssem_b, isem0, isem1):
            _sc_agg_body(False, x_hbm, src_hbm, dst_hbm, None, zagg_hbm,
                         part_hbm, None, agg_sh, src_v0, src_v1, dst_v0,
                         dst_v1, buf_a, buf_b, gsem_a, gsem_b, ssem_a,
                         ssem_b, isem0, isem1)
    return pl.kernel(body, out_type=tuple(out_type), mesh=_MESH,
                     scratch_types=scratch)


_sc_agg_deg = _make_sc_agg(True)
_sc_agg = _make_sc_agg(False)


def _tc_layer_body(final, x_ref, p_ref, degp_ref, W1_ref, b1_ref, g1_ref,
                   bt1_ref, W2_ref, b2_ref, g2_ref, bt2_ref, *out_refs):
    deg = degp_ref[0, :N, 0:1] + degp_ref[1, :N, 0:1]
    recip = 1.0 / jnp.maximum(deg, 1.0)
    h = x_ref[...] + (p_ref[0, :N] + p_ref[1, :N]) * recip
    t = jnp.dot(h, W1_ref[...], preferred_element_type=jnp.float32) + b1_ref[...]
    m = jnp.mean(t, axis=0)
    v = jnp.mean((t - m) ** 2, axis=0)
    t = jnp.maximum((t - m) * lax.rsqrt(v + 1e-5) * g1_ref[...] + bt1_ref[...], 0.0)
    t = jnp.dot(t, W2_ref[...], preferred_element_type=jnp.float32) + b2_ref[...]
    m = jnp.mean(t, axis=0)
    v = jnp.mean((t - m) ** 2, axis=0)
    t = jnp.maximum((t - m) * lax.rsqrt(v + 1e-5) * g2_ref[...] + bt2_ref[...], 0.0)
    out_refs[0][...] = t
    if final:
        out_refs[1][...] = jnp.mean(t, axis=0, keepdims=True)


def _make_tc_layer(final):
    outs = [jax.ShapeDtypeStruct((N, D), jnp.float32)]
    if final:
        outs.append(jax.ShapeDtypeStruct((1, D), jnp.float32))
    return pl.pallas_call(
        functools.partial(_tc_layer_body, final),
        out_shape=tuple(outs),
    )


_tc_layer = _make_tc_layer(False)
_tc_layer_final = _make_tc_layer(True)


def kernel(features, edge_index, l0_W1, l0_b1, l0_g1, l0_bt1, l0_W2, l0_b2,
           l0_g2, l0_bt2, l1_W1, l1_b1, l1_g1, l1_bt1, l1_W2, l1_b2, l1_g2,
           l1_bt2):
    src = edge_index[0]
    dst = edge_index[1]
    half_e = E // 2
    pad_half = E_PAD // 2 - half_e
    zpad = jnp.zeros((pad_half,), jnp.int32)
    tpad = N + jnp.arange(pad_half, dtype=jnp.int32) % (NPAD - N)
    src_p = jnp.concatenate([src[:half_e], zpad, src[half_e:], zpad]).reshape(-1, CHUNK)
    dst_p = jnp.concatenate([dst[:half_e], tpad, dst[half_e:], tpad]).reshape(-1, CHUNK)
    ones = jnp.ones((CHUNK, D), jnp.float32)
    zagg = jnp.zeros((ZROWS, D), jnp.float32)

    parts0, degp = _sc_agg_deg(features, src_p, dst_p, ones, zagg)
    x1 = _tc_layer(features, parts0, degp, l0_W1, l0_b1, l0_g1, l0_bt1,
                   l0_W2, l0_b2, l0_g2, l0_bt2)[0]
    parts1 = _sc_agg(x1, src_p, dst_p, zagg)[0]
    x2, pool = _tc_layer_final(x1, parts1, degp, l1_W1, l1_b1, l1_g1, l1_bt1,
                               l1_W2, l1_b2, l1_g2, l1_bt2)
    return (pool, x2)
